# baseline (speedup 1.0000x reference)
import jax
import jax.numpy as jnp
from jax import lax
from jax.experimental import pallas as pl
from jax.experimental.pallas import tpu as pltpu
from jax.experimental.pallas import tpu_sc as plsc

_VOCAB = 1000000
_MAXLEN = 200
_EMBED = 32
_BATCH = 4096

_NW = 32
_BPW = _BATCH // _NW
_IDXW = _BPW * _MAXLEN
_JB = _EMBED // 8


def _sc_body(x_hbm, tok_hbm, pos_hbm, out_hbm,
             x_loc, x_t, g0, g1, s0, s1, posf, gs0, gs1, os0, os1):
    cid = lax.axis_index("c")
    sid = lax.axis_index("s")
    wid = sid * 2 + cid

    pltpu.sync_copy(x_hbm.at[pl.ds(wid * _IDXW, _IDXW)], x_loc)
    pltpu.sync_copy(pos_hbm, posf)

    iota16 = lax.broadcasted_iota(jnp.int32, (16,), 0)
    zeros16 = jnp.zeros((16,), jnp.int32)
    iota_x = iota16 * _MAXLEN
    row_idx = [iota16 + b0 * 16 for b0 in range(8)]

    def tbody(p, carry):
        for b0 in range(8):
            idx = iota_x + (b0 * 16 * _MAXLEN + p)
            x_t[p, pl.ds(b0 * 16, 16)] = plsc.load_gather(x_loc, [idx])
        return carry
    lax.fori_loop(0, _MAXLEN, tbody, 0)

    g = (g0, g1)
    s = (s0, s1)
    gsem = (gs0, gs1)
    osem = (os0, os1)

    def fire_gather(p, b):
        pltpu.async_copy(tok_hbm.at[x_t.at[p]], g[b], gsem[b])

    def wait_gather(b):
        pltpu.make_async_copy(tok_hbm.at[x_t.at[0]], g[b], gsem[b]).wait()

    def fire_out(p, b):
        for jb in range(_JB):
            pltpu.async_copy(s[b].at[jb], out_hbm.at[p, jb, wid], osem[b])

    def wait_out(b):
        for jb in range(_JB):
            pltpu.make_async_copy(s[b].at[jb], out_hbm.at[0, jb, wid],
                                  osem[b]).wait()

    col_idx = [zeros16 + j for j in range(_EMBED)]

    def compute(p, b):
        p32 = p * _EMBED
        for j in range(_EMBED):
            pos_splat = plsc.load_gather(posf, [zeros16 + (p32 + j)])
            for b0 in range(8):
                v = plsc.load_gather(g[b], [row_idx[b0], col_idx[j]])
                s[b][j // 8, j % 8, pl.ds(b0 * 16, 16)] = v + pos_splat

    fire_gather(0, 0)

    def outer(p2, carry):
        for b in range(2):
            p = p2 * 2 + b
            nxt = p + 1

            @pl.when(nxt < _MAXLEN)
            def _prefetch():
                fire_gather(nxt, 1 - b)

            wait_gather(b)

            @pl.when(p >= 2)
            def _drain():
                wait_out(b)

            compute(p, b)
            fire_out(p, b)
        return carry

    lax.fori_loop(0, _MAXLEN // 2, outer, 0)
    wait_out(0)
    wait_out(1)


def kernel(x, token_table, pos_table):
    x_flat = x.reshape(-1).astype(jnp.int32)
    pos_flat = pos_table.reshape(-1)
    mesh = plsc.VectorSubcoreMesh(core_axis_name="c", subcore_axis_name="s")
    f = pl.kernel(
        _sc_body,
        out_type=jax.ShapeDtypeStruct((_MAXLEN, _JB, _NW, 8, 128),
                                      jnp.float32),
        mesh=mesh,
        compiler_params=pltpu.CompilerParams(use_tc_tiling_on_sc=False,
                                             needs_layout_passes=False),
        scratch_types=[
            pltpu.VMEM((_IDXW,), jnp.int32),
            pltpu.VMEM((_MAXLEN, _BPW), jnp.int32),
            pltpu.VMEM((_BPW, _EMBED), jnp.float32),
            pltpu.VMEM((_BPW, _EMBED), jnp.float32),
            pltpu.VMEM((_JB, 8, 128), jnp.float32),
            pltpu.VMEM((_JB, 8, 128), jnp.float32),
            pltpu.VMEM((_MAXLEN * _EMBED,), jnp.float32),
            pltpu.SemaphoreType.DMA,
            pltpu.SemaphoreType.DMA,
            pltpu.SemaphoreType.DMA,
            pltpu.SemaphoreType.DMA,
        ],
    )
    out5 = f(x_flat, token_table, pos_flat)
    return out5.transpose(2, 4, 0, 1, 3).reshape(_BATCH, _MAXLEN, _EMBED)


# speedup vs baseline: 1.5677x; 1.5677x over previous
import jax
import jax.numpy as jnp
from jax import lax
from jax.experimental import pallas as pl
from jax.experimental.pallas import tpu as pltpu
from jax.experimental.pallas import tpu_sc as plsc

_VOCAB = 1000000
_MAXLEN = 200
_EMBED = 32
_BATCH = 4096

_NW = 32
_BPW = _BATCH // _NW
_IDXW = _BPW * _MAXLEN
_JB = _EMBED // 8


def _sc_body(x_hbm, tok_hbm, pos_hbm, out_hbm,
             x_loc, x_t, g0, g1, s0, s1, posf, gs0, gs1, os0, os1):
    cid = lax.axis_index("c")
    sid = lax.axis_index("s")
    wid = sid * 2 + cid

    pltpu.sync_copy(x_hbm.at[pl.ds(wid * _IDXW, _IDXW)], x_loc)
    pltpu.sync_copy(pos_hbm, posf)

    iota16 = lax.broadcasted_iota(jnp.int32, (16,), 0)
    zeros16 = jnp.zeros((16,), jnp.int32)
    iota_x = iota16 * _MAXLEN
    row_idx = [iota16 + b0 * 16 for b0 in range(8)]

    def tbody(p, carry):
        for b0 in range(8):
            idx = iota_x + (b0 * 16 * _MAXLEN + p)
            x_t[p, pl.ds(b0 * 16, 16)] = plsc.load_gather(x_loc, [idx])
        return carry
    lax.fori_loop(0, _MAXLEN, tbody, 0)

    g = (g0, g1)
    s = (s0, s1)
    gsem = (gs0, gs1)
    osem = (os0, os1)

    def fire_gather(p, b):
        pltpu.async_copy(tok_hbm.at[x_t.at[p]], g[b], gsem[b])

    def wait_gather(b):
        pltpu.make_async_copy(tok_hbm.at[x_t.at[0]], g[b], gsem[b]).wait()

    def fire_out(p, b):
        for jb in range(_JB):
            pltpu.async_copy(s[b].at[pl.ds(jb * 8, 8), pl.ds(0, 128)],
                             out_hbm.at[p, jb, wid], osem[b])

    def wait_out(b):
        for jb in range(_JB):
            pltpu.make_async_copy(s[b].at[pl.ds(jb * 8, 8), pl.ds(0, 128)],
                                  out_hbm.at[0, jb, wid], osem[b]).wait()

    row_lo = iota16
    row_hi = iota16 + 16

    def compute(p, b):
        p32 = p * _EMBED
        pos0 = posf[pl.ds(p32, 16)]
        pos1 = posf[pl.ds(p32 + 16, 16)]
        for bi in range(_BPW):
            colv = zeros16 + bi
            v0 = g[b][bi, pl.ds(0, 16)] + pos0
            v1 = g[b][bi, pl.ds(16, 16)] + pos1
            plsc.store_scatter(s[b], [row_lo, colv], v0)
            plsc.store_scatter(s[b], [row_hi, colv], v1)

    fire_gather(0, 0)

    def outer(p2, carry):
        for b in range(2):
            p = p2 * 2 + b
            nxt = p + 1

            @pl.when(nxt < _MAXLEN)
            def _prefetch():
                fire_gather(nxt, 1 - b)

            wait_gather(b)

            @pl.when(p >= 2)
            def _drain():
                wait_out(b)

            compute(p, b)
            fire_out(p, b)
        return carry

    lax.fori_loop(0, _MAXLEN // 2, outer, 0)
    wait_out(0)
    wait_out(1)


def kernel(x, token_table, pos_table):
    x_flat = x.reshape(-1).astype(jnp.int32)
    pos_flat = pos_table.reshape(-1)
    mesh = plsc.VectorSubcoreMesh(core_axis_name="c", subcore_axis_name="s")
    f = pl.kernel(
        _sc_body,
        out_type=jax.ShapeDtypeStruct((_MAXLEN, _JB, _NW, 8, 128),
                                      jnp.float32),
        mesh=mesh,
        compiler_params=pltpu.CompilerParams(use_tc_tiling_on_sc=False,
                                             needs_layout_passes=False),
        scratch_types=[
            pltpu.VMEM((_IDXW,), jnp.int32),
            pltpu.VMEM((_MAXLEN, _BPW), jnp.int32),
            pltpu.VMEM((_BPW, _EMBED), jnp.float32),
            pltpu.VMEM((_BPW, _EMBED), jnp.float32),
            pltpu.VMEM((_EMBED, 129), jnp.float32),
            pltpu.VMEM((_EMBED, 129), jnp.float32),
            pltpu.VMEM((_MAXLEN * _EMBED,), jnp.float32),
            pltpu.SemaphoreType.DMA,
            pltpu.SemaphoreType.DMA,
            pltpu.SemaphoreType.DMA,
            pltpu.SemaphoreType.DMA,
        ],
    )
    out5 = f(x_flat, token_table, pos_flat)
    return out5.transpose(2, 4, 0, 1, 3).reshape(_BATCH, _MAXLEN, _EMBED)


# speedup vs baseline: 1.5715x; 1.0024x over previous
import jax
import jax.numpy as jnp
from jax import lax
from jax.experimental import pallas as pl
from jax.experimental.pallas import tpu as pltpu
from jax.experimental.pallas import tpu_sc as plsc

_VOCAB = 1000000
_MAXLEN = 200
_EMBED = 32
_BATCH = 4096

_NW = 32
_BPW = _BATCH // _NW
_IDXW = _BPW * _MAXLEN
_JB = _EMBED // 8


def _sc_body(x_hbm, tok_hbm, pos_hbm, out_hbm,
             x_t, g0, g1, s0, s1, posf, gs0, gs1, os0, os1):
    cid = lax.axis_index("c")
    sid = lax.axis_index("s")
    wid = sid * 2 + cid

    pltpu.sync_copy(x_hbm.at[:, wid], x_t)
    pltpu.sync_copy(pos_hbm, posf)

    iota16 = lax.broadcasted_iota(jnp.int32, (16,), 0)
    zeros16 = jnp.zeros((16,), jnp.int32)

    g = (g0, g1)
    s = (s0, s1)
    gsem = (gs0, gs1)
    osem = (os0, os1)

    def fire_gather(p, b):
        pltpu.async_copy(tok_hbm.at[x_t.at[p // 8, p % 8]], g[b], gsem[b])

    def wait_gather(b):
        pltpu.make_async_copy(tok_hbm.at[x_t.at[0, 0]], g[b], gsem[b]).wait()

    def fire_out(p, b):
        for jb in range(_JB):
            pltpu.async_copy(s[b].at[pl.ds(jb * 8, 8), pl.ds(0, 128)],
                             out_hbm.at[p, jb, wid], osem[b])

    def wait_out(b):
        for jb in range(_JB):
            pltpu.make_async_copy(s[b].at[pl.ds(jb * 8, 8), pl.ds(0, 128)],
                                  out_hbm.at[0, jb, wid], osem[b]).wait()

    row_lo = iota16
    row_hi = iota16 + 16

    def compute(p, b):
        p32 = p * _EMBED
        pos0 = posf[pl.ds(p32, 16)]
        pos1 = posf[pl.ds(p32 + 16, 16)]
        for bi in range(_BPW):
            colv = zeros16 + bi
            v0 = g[b][bi, pl.ds(0, 16)] + pos0
            v1 = g[b][bi, pl.ds(16, 16)] + pos1
            plsc.store_scatter(s[b], [row_lo, colv], v0)
            plsc.store_scatter(s[b], [row_hi, colv], v1)

    fire_gather(0, 0)

    def outer(p2, carry):
        for b in range(2):
            p = p2 * 2 + b
            nxt = p + 1

            @pl.when(nxt < _MAXLEN)
            def _prefetch():
                fire_gather(nxt, 1 - b)

            wait_gather(b)

            @pl.when(p >= 2)
            def _drain():
                wait_out(b)

            compute(p, b)
            fire_out(p, b)
        return carry

    lax.fori_loop(0, _MAXLEN // 2, outer, 0)
    wait_out(0)
    wait_out(1)


def kernel(x, token_table, pos_table):
    x5 = (x.astype(jnp.int32)
          .reshape(_NW, 128, _MAXLEN // 8, 8)
          .transpose(2, 0, 3, 1))
    pos_flat = pos_table.reshape(-1)
    mesh = plsc.VectorSubcoreMesh(core_axis_name="c", subcore_axis_name="s")
    f = pl.kernel(
        _sc_body,
        out_type=jax.ShapeDtypeStruct((_MAXLEN, _JB, _NW, 8, 128),
                                      jnp.float32),
        mesh=mesh,
        compiler_params=pltpu.CompilerParams(use_tc_tiling_on_sc=False,
                                             needs_layout_passes=False),
        scratch_types=[
            pltpu.VMEM((_MAXLEN // 8, 8, _BPW), jnp.int32),
            pltpu.VMEM((_BPW, _EMBED), jnp.float32),
            pltpu.VMEM((_BPW, _EMBED), jnp.float32),
            pltpu.VMEM((_EMBED, 129), jnp.float32),
            pltpu.VMEM((_EMBED, 129), jnp.float32),
            pltpu.VMEM((_MAXLEN * _EMBED,), jnp.float32),
            pltpu.SemaphoreType.DMA,
            pltpu.SemaphoreType.DMA,
            pltpu.SemaphoreType.DMA,
            pltpu.SemaphoreType.DMA,
        ],
    )
    out5 = f(x5, token_table, pos_flat)
    return out5.transpose(2, 4, 0, 1, 3).reshape(_BATCH, _MAXLEN, _EMBED)
